# Initial kernel scaffold; baseline (speedup 1.0000x reference)
import jax
import jax.numpy as jnp
from jax.experimental import pallas as pl


def kernel(t, data, edges, W1, b1, W2, b2):
    raise NotImplementedError("write your pallas kernel here")


# speedup vs baseline: 5.5516x; 5.5516x over previous
import jax
import jax.numpy as jnp
from jax.experimental import pallas as pl
from jax.experimental.pallas import tpu as pltpu

_SENT = (1 << 14) - 1


def _sort4(c0, c1, c2, c3):
    a, b = jnp.minimum(c0, c1), jnp.maximum(c0, c1)
    c, d = jnp.minimum(c2, c3), jnp.maximum(c2, c3)
    e, f = jnp.minimum(a, c), jnp.maximum(a, c)
    g, h = jnp.minimum(b, d), jnp.maximum(b, d)
    i, k = jnp.minimum(f, g), jnp.maximum(f, g)
    return e, i, k, h


def _kernel_a(t_ref, dst_ref, data_ref, wcat_ref, w0_ref, b1_ref,
              base_ref, extra_ref, neigh_scr, extra_scr, *, blk, nblk):
    j = pl.program_id(0)

    @pl.when(j == 0)
    def _():
        d = dst_ref[...]
        s0, s1, s2, s3 = _sort4(d[:, 0:1], d[:, 1:2], d[:, 2:3], d[:, 3:4])
        d1 = jnp.where(s1 == s0, _SENT, s1)
        d2 = jnp.where(s2 == s1, _SENT, s2)
        d3 = jnp.where(s3 == s2, _SENT, s3)
        n0, n1, n2, n3 = _sort4(s0, d1, d2, d3)
        neigh_scr[...] = jnp.concatenate([n0, n1, n2, n3], axis=1)
        extra_scr[...] = jnp.zeros_like(extra_scr)

    blkd = data_ref[...]
    p = jnp.dot(blkd, wcat_ref[...], preferred_element_type=jnp.float32)
    tvec = t_ref[0] * w0_ref[...] + b1_ref[...]
    base_ref[...] = p[:, 0:16] + tvec

    rowid = j * blk + jax.lax.broadcasted_iota(jnp.int32, (1, blk), 1)
    ng = neigh_scr[...]
    acc = extra_scr[...]
    for k in range(4):
        m = (ng[:, k:k + 1] == rowid).astype(jnp.float32)
        acc = acc + jnp.dot(m, p[:, 16 * (k + 1):16 * (k + 2)],
                            preferred_element_type=jnp.float32)
    extra_scr[...] = acc

    @pl.when(j == nblk - 1)
    def _():
        extra_ref[...] = extra_scr[...]


def _kernel_b(src_ref, base_ref, extra_ref, w2_ref, b2_ref, out_ref, *, blk):
    j = pl.program_id(0)
    rowid = j * blk + jax.lax.broadcasted_iota(jnp.int32, (blk, 1), 0)
    oh = (rowid == src_ref[...]).astype(jnp.float32)
    pre = base_ref[...] + jnp.dot(oh, extra_ref[...],
                                  preferred_element_type=jnp.float32)
    h = jnp.tanh(pre)
    out_ref[...] = jnp.dot(h, w2_ref[...],
                           preferred_element_type=jnp.float32) + b2_ref[...]


def kernel(t, data, edges, W1, b1, W2, b2):
    n, c = data.shape
    e = edges.shape[1]
    s = e // 4
    sp = 512
    blk = 1000
    nblk = n // blk

    src = edges[0].astype(jnp.int32).reshape(s, 4)[:, 0]
    dst = edges[1].astype(jnp.int32).reshape(s, 4)
    srcp = jnp.pad(src, (0, sp - s), constant_values=-1).reshape(1, sp)
    dstp = jnp.pad(dst, ((0, sp - s), (0, 0)), constant_values=_SENT)

    w1p = jnp.pad(W1, ((0, 0), (0, 1)))
    w0 = w1p[0:1]
    a_mat = w1p[1:1 + c]
    b_mats = [w1p[1 + c * (k + 1):1 + c * (k + 2)] for k in range(4)]
    wcat = jnp.concatenate([a_mat] + b_mats, axis=1)
    b1p = jnp.pad(b1, (0, 1)).reshape(1, 16)
    w2p = jnp.pad(W2, ((0, 1), (0, 0)))
    b2r = b2.reshape(1, c)
    tt = t.astype(jnp.float32)

    import functools
    base, extra = pl.pallas_call(
        functools.partial(_kernel_a, blk=blk, nblk=nblk),
        grid=(nblk,),
        in_specs=[
            pl.BlockSpec(memory_space=pltpu.SMEM),
            pl.BlockSpec((sp, 4), lambda j: (0, 0)),
            pl.BlockSpec((blk, c), lambda j: (j, 0)),
            pl.BlockSpec((c, 80), lambda j: (0, 0)),
            pl.BlockSpec((1, 16), lambda j: (0, 0)),
            pl.BlockSpec((1, 16), lambda j: (0, 0)),
        ],
        out_specs=[
            pl.BlockSpec((blk, 16), lambda j: (j, 0)),
            pl.BlockSpec((sp, 16), lambda j: (0, 0)),
        ],
        out_shape=[
            jax.ShapeDtypeStruct((n, 16), jnp.float32),
            jax.ShapeDtypeStruct((sp, 16), jnp.float32),
        ],
        scratch_shapes=[
            pltpu.VMEM((sp, 4), jnp.int32),
            pltpu.VMEM((sp, 16), jnp.float32),
        ],
    )(tt, dstp, data, wcat, w0, b1p)

    out = pl.pallas_call(
        functools.partial(_kernel_b, blk=blk),
        grid=(nblk,),
        in_specs=[
            pl.BlockSpec((1, sp), lambda j: (0, 0)),
            pl.BlockSpec((blk, 16), lambda j: (j, 0)),
            pl.BlockSpec((sp, 16), lambda j: (0, 0)),
            pl.BlockSpec((16, c), lambda j: (0, 0)),
            pl.BlockSpec((1, c), lambda j: (0, 0)),
        ],
        out_specs=pl.BlockSpec((blk, c), lambda j: (j, 0)),
        out_shape=jax.ShapeDtypeStruct((n, c), jnp.float32),
    )(srcp, base, extra, w2p, b2r)
    return out
